# baseline (speedup 1.0000x reference)
import functools

import jax
import jax.numpy as jnp
from jax import lax
from jax.experimental import pallas as pl
from jax.experimental.pallas import tpu as pltpu
from jax.experimental.pallas import tpu_sc as plsc

_NC, _NS = 2, 16
_NW = _NC * _NS


def _sc_body(x_hbm, sbc_hbm, out_hbm, buf, svec, B, C, H, W, RC):
    wid = lax.axis_index("s") * _NC + lax.axis_index("c")
    spw = B // _NW
    nrb = H // RC
    lanes_per_row = W // 16
    for t in range(spw):
        b = wid * spw + t
        pltpu.sync_copy(sbc_hbm.at[b], svec)
        sv = svec[...]
        for cc in range(C):
            for rb in range(nrb):
                pltpu.sync_copy(x_hbm.at[b, cc, pl.ds(rb * RC, RC), :], buf)

                def row_body(i, _, sv=sv):
                    for j in range(lanes_per_row):
                        sl = pl.ds(j * 16, 16)
                        buf[i, sl] = buf[i, sl] * sv
                    return 0

                lax.fori_loop(0, RC, row_body, 0)
                pltpu.sync_copy(buf, out_hbm.at[b, cc, pl.ds(rb * RC, RC), :])


def kernel(input, probs, magnitudes):
    B, C, H, W = input.shape
    scale = jnp.where(probs, magnitudes, jnp.float32(1.0))
    sbc = jnp.broadcast_to(scale[:, None], (B, 16))
    RC = 128
    body = functools.partial(_sc_body, B=B, C=C, H=H, W=W, RC=RC)
    k = pl.kernel(
        body,
        out_type=jax.ShapeDtypeStruct((B, C, H, W), jnp.float32),
        mesh=plsc.VectorSubcoreMesh(core_axis_name="c", subcore_axis_name="s"),
        scratch_types=[
            pltpu.VMEM((RC, W), jnp.float32),
            pltpu.VMEM((16,), jnp.float32),
        ],
    )
    return k(input, sbc)
